# Initial kernel scaffold; baseline (speedup 1.0000x reference)
import jax
import jax.numpy as jnp
from jax.experimental import pallas as pl


def kernel(x_0, x_1, x_2, neighborhood_0_to_0, neighborhood_1_to_1, neighborhood_2_to_2, neighborhood_0_to_1, neighborhood_1_to_2, params):
    raise NotImplementedError("write your pallas kernel here")


# speedup vs baseline: 1.3857x; 1.3857x over previous
import functools

import jax
import jax.numpy as jnp
from jax.experimental import pallas as pl
from jax.experimental.pallas import tpu as pltpu

C = 128
NEG = -1e9


def _leaky(x, slope):
    return jnp.where(x > 0, x, slope * x)


def _proj_body(n_add, relu_in, *refs):
    xs = refs[:n_add]
    w_ref, au_ref, av_ref, m_ref, u_ref, vt_ref = refs[n_add:]
    x = xs[0][...]
    for r in xs[1:]:
        x = x + r[...]
    if relu_in:
        x = jnp.maximum(x, 0.0)
    m = jnp.dot(x, w_ref[...], preferred_element_type=jnp.float32)
    m_ref[...] = m
    u_ref[...] = jnp.dot(m, au_ref[...], preferred_element_type=jnp.float32)
    vt_ref[...] = jax.lax.dot_general(
        av_ref[...], m, (((0,), (1,)), ((), ())),
        preferred_element_type=jnp.float32)


def _proj(xs, w, au, av, relu_in):
    n = xs[0].shape[0]
    body = functools.partial(_proj_body, len(xs), relu_in)
    return pl.pallas_call(
        body,
        out_shape=[
            jax.ShapeDtypeStruct((n, C), jnp.float32),
            jax.ShapeDtypeStruct((n, 1), jnp.float32),
            jax.ShapeDtypeStruct((1, n), jnp.float32),
        ],
    )(*xs, w, au, av)


def _hbs_body(A_ref, u_ref, vt_ref, m_ref, o_ref):
    A = A_ref[...]
    mask = A != 0
    e = _leaky(u_ref[...] + vt_ref[...], 0.2)
    e = jnp.where(mask, e, NEG)
    mx = jnp.max(e, axis=1, keepdims=True)
    ex = jnp.where(mask, jnp.exp(e - mx), 0.0)
    s = jnp.maximum(jnp.sum(ex, axis=1, keepdims=True), 1e-13)
    w = A * ex / s
    o = jnp.dot(w, m_ref[...], preferred_element_type=jnp.float32)
    o_ref[...] = jnp.maximum(o, 0.0)


def _hbs_attn(A, u, vt, m, tile):
    N, S = A.shape
    return pl.pallas_call(
        _hbs_body,
        grid=(N // tile,),
        in_specs=[
            pl.BlockSpec((tile, S), lambda i: (i, 0)),
            pl.BlockSpec((tile, 1), lambda i: (i, 0)),
            pl.BlockSpec((1, S), lambda i: (0, 0)),
            pl.BlockSpec((S, C), lambda i: (0, 0)),
        ],
        out_specs=pl.BlockSpec((tile, C), lambda i: (i, 0)),
        out_shape=jax.ShapeDtypeStruct((N, C), jnp.float32),
    )(A, u, vt, m)


def _hbns_body(A_ref, u_ref, vt_ref, sm_ref, tm_ref, ot_ref, os_ref,
               cmax_ref, csum_ref, acc_ref):
    i = pl.program_id(0)
    nt = pl.num_programs(0)
    A = A_ref[...]
    mask = A != 0
    e = _leaky(u_ref[...] + vt_ref[...], 0.2)
    e = jnp.where(mask, e, NEG)

    mx = jnp.max(e, axis=1, keepdims=True)
    ex = jnp.where(mask, jnp.exp(e - mx), 0.0)
    s = jnp.maximum(jnp.sum(ex, axis=1, keepdims=True), 1e-13)
    w1 = A * ex / s
    ot_ref[...] = jnp.maximum(
        jnp.dot(w1, sm_ref[...], preferred_element_type=jnp.float32), 0.0)

    @pl.when(i == 0)
    def _():
        cmax_ref[...] = jnp.full(cmax_ref.shape, NEG, jnp.float32)
        csum_ref[...] = jnp.zeros(csum_ref.shape, jnp.float32)
        acc_ref[...] = jnp.zeros(acc_ref.shape, jnp.float32)

    tmx = jnp.max(e, axis=0, keepdims=True)
    new = jnp.maximum(cmax_ref[...], tmx)
    scale = jnp.exp(cmax_ref[...] - new)
    ex2 = jnp.where(mask, jnp.exp(e - new), 0.0)
    csum_ref[...] = csum_ref[...] * scale + jnp.sum(ex2, axis=0, keepdims=True)
    w2 = A * ex2
    pacc = jax.lax.dot_general(
        w2, tm_ref[...], (((0,), (0,)), ((), ())),
        preferred_element_type=jnp.float32)
    acc_ref[...] = acc_ref[...] * scale.T + pacc
    cmax_ref[...] = new

    @pl.when(i == nt - 1)
    def _():
        den = jnp.maximum(csum_ref[...], 1e-13)
        os_ref[...] = jnp.maximum(acc_ref[...] / den.T, 0.0)


def _hbns_attn(A, u, vt, sm, tm, tile):
    T, S = A.shape
    return pl.pallas_call(
        _hbns_body,
        grid=(T // tile,),
        in_specs=[
            pl.BlockSpec((tile, S), lambda i: (i, 0)),
            pl.BlockSpec((tile, 1), lambda i: (i, 0)),
            pl.BlockSpec((1, S), lambda i: (0, 0)),
            pl.BlockSpec((S, C), lambda i: (0, 0)),
            pl.BlockSpec((tile, C), lambda i: (i, 0)),
        ],
        out_specs=[
            pl.BlockSpec((tile, C), lambda i: (i, 0)),
            pl.BlockSpec((S, C), lambda i: (0, 0)),
        ],
        out_shape=[
            jax.ShapeDtypeStruct((T, C), jnp.float32),
            jax.ShapeDtypeStruct((S, C), jnp.float32),
        ],
        scratch_shapes=[
            pltpu.VMEM((1, S), jnp.float32),
            pltpu.VMEM((1, S), jnp.float32),
            pltpu.VMEM((S, C), jnp.float32),
        ],
    )(A, u, vt, sm, tm)


def _head_body(a0_ref, b0_ref, a1_ref, b1_ref, c1_ref, a2_ref, b2_ref,
               w1_ref, bb1_ref, w2_ref, bb2_ref, w3_ref, bb3_ref,
               w4_ref, bb4_ref, o_ref):
    x0 = jnp.maximum(a0_ref[...] + b0_ref[...], 0.0)
    p0 = jnp.max(x0, axis=0, keepdims=True)
    x1 = jnp.maximum(a1_ref[...] + b1_ref[...] + c1_ref[...], 0.0)
    p1 = jnp.max(x1, axis=0, keepdims=True)
    x2 = jnp.maximum(a2_ref[...] + b2_ref[...], 0.0)
    p2 = jnp.max(x2, axis=0, keepdims=True)
    w1 = w1_ref[...]
    h = (jnp.dot(p0, w1[0:C], preferred_element_type=jnp.float32)
         + jnp.dot(p1, w1[C:2 * C], preferred_element_type=jnp.float32)
         + jnp.dot(p2, w1[2 * C:3 * C], preferred_element_type=jnp.float32)
         + bb1_ref[...])
    h = _leaky(h, 0.01)
    h = _leaky(jnp.dot(h, w2_ref[...], preferred_element_type=jnp.float32)
               + bb2_ref[...], 0.01)
    h = _leaky(jnp.dot(h, w3_ref[...], preferred_element_type=jnp.float32)
               + bb3_ref[...], 0.01)
    o_ref[...] = (jnp.dot(h, w4_ref[...], preferred_element_type=jnp.float32)
                  + bb4_ref[...])


def _head(msgs, p):
    out_dim = p["fc4_b"].shape[0]
    return pl.pallas_call(
        _head_body,
        out_shape=jax.ShapeDtypeStruct((1, out_dim), jnp.float32),
    )(*msgs,
      p["fc1_w"], p["fc1_b"].reshape(1, -1),
      p["fc2_w"], p["fc2_b"].reshape(1, -1),
      p["fc3_w"], p["fc3_b"].reshape(1, -1),
      p["fc4_w"], p["fc4_b"].reshape(1, -1))


def _halves(a):
    return a[:C].reshape(C, 1), a[C:].reshape(C, 1)


def kernel(x_0, x_1, x_2, neighborhood_0_to_0, neighborhood_1_to_1,
           neighborhood_2_to_2, neighborhood_0_to_1, neighborhood_1_to_2,
           params):
    p = params
    n00, n11, n22 = neighborhood_0_to_0, neighborhood_1_to_1, neighborhood_2_to_2
    n01, n12 = neighborhood_0_to_1, neighborhood_1_to_2
    TT = 200

    au, av = _halves(p["hbs0_l1_a"])
    m0, u0, v0t = _proj((x_0,), p["hbs0_l1_w"], au, av, False)
    x00 = _hbs_attn(n00, u0, v0t, m0, TT)

    asrc, atgt = _halves(p["hbns01_l1_a"])
    tm01, ut01, _ = _proj((x_0,), p["hbns01_l1_wt"], atgt, atgt, False)
    sm01, _, vst01 = _proj((x_1,), p["hbns01_l1_ws"], asrc, asrc, False)
    x1to0, x0to1 = _hbns_attn(n01, ut01, vst01, sm01, tm01, TT)

    asrc, atgt = _halves(p["hbns12_l1_a"])
    tm12, ut12, _ = _proj((x_1,), p["hbns12_l1_wt"], atgt, atgt, False)
    sm12, _, vst12 = _proj((x_2,), p["hbns12_l1_ws"], asrc, asrc, False)
    x2to1, x1to2 = _hbns_attn(n12, ut12, vst12, sm12, tm12, TT)

    x0l1 = (x00, x1to0)
    x1l1 = (x0to1, x2to1)
    x2l1 = (x1to2,)

    au, av = _halves(p["hbs0_l2_a"])
    m0b, u0b, v0bt = _proj(x0l1, p["hbs0_l2_w"], au, av, True)
    x00_2 = _hbs_attn(n00, u0b, v0bt, m0b, TT)

    asrc, atgt = _halves(p["hbns01_l2_a"])
    tm01b, ut01b, _ = _proj(x0l1, p["hbns01_l2_wt"], atgt, atgt, True)
    sm01b, _, vst01b = _proj(x1l1, p["hbns01_l2_ws"], asrc, asrc, True)
    x1to0_2, x0to1_2 = _hbns_attn(n01, ut01b, vst01b, sm01b, tm01b, TT)

    au, av = _halves(p["hbs1_l2_a"])
    m1b, u1b, v1bt = _proj(x1l1, p["hbs1_l2_w"], au, av, True)
    x11 = _hbs_attn(n11, u1b, v1bt, m1b, TT)

    asrc, atgt = _halves(p["hbns12_l2_a"])
    tm12b, ut12b, _ = _proj(x1l1, p["hbns12_l2_wt"], atgt, atgt, True)
    sm12b, _, vst12b = _proj(x2l1, p["hbns12_l2_ws"], asrc, asrc, True)
    x2to1_2, x1to2_2 = _hbns_attn(n12, ut12b, vst12b, sm12b, tm12b, TT)

    au, av = _halves(p["hbs2_l2_a"])
    m2b, u2b, v2bt = _proj(x2l1, p["hbs2_l2_w"], au, av, True)
    x22 = _hbs_attn(n22, u2b, v2bt, m2b, TT)

    return _head((x00_2, x1to0_2, x0to1_2, x11, x2to1_2, x1to2_2, x22), p)


# speedup vs baseline: 1.4954x; 1.0792x over previous
import functools

import jax
import jax.numpy as jnp
from jax.experimental import pallas as pl
from jax.experimental.pallas import tpu as pltpu

C = 128
NEG = -1e9


def _leaky(x, slope):
    return jnp.where(x > 0, x, slope * x)


def _projs_body(specs, *refs):
    pos = 0
    ins = []
    for n_add, relu_in, want_u, want_v in specs:
        ins.append(refs[pos:pos + n_add + 3])
        pos += n_add + 3
    outs = refs[pos:]
    opos = 0
    for (n_add, relu_in, want_u, want_v), group in zip(specs, ins):
        xs = group[:n_add]
        w_ref, au_ref, av_ref = group[n_add:]
        x = xs[0][...]
        for r in xs[1:]:
            x = x + r[...]
        if relu_in:
            x = jnp.maximum(x, 0.0)
        m = jnp.dot(x, w_ref[...], preferred_element_type=jnp.float32)
        outs[opos][...] = m
        opos += 1
        if want_u:
            outs[opos][...] = jnp.dot(m, au_ref[...],
                                      preferred_element_type=jnp.float32)
            opos += 1
        if want_v:
            outs[opos][...] = jax.lax.dot_general(
                av_ref[...], m, (((0,), (1,)), ((), ())),
                preferred_element_type=jnp.float32)
            opos += 1


def _projs(groups):
    specs = tuple((len(xs), relu_in, want_u, want_v)
                  for xs, _, _, _, relu_in, want_u, want_v in groups)
    operands = []
    out_shapes = []
    for xs, w, au, av, relu_in, want_u, want_v in groups:
        n = xs[0].shape[0]
        operands.extend(xs)
        operands.extend((w, au, av))
        out_shapes.append(jax.ShapeDtypeStruct((n, C), jnp.float32))
        if want_u:
            out_shapes.append(jax.ShapeDtypeStruct((n, 1), jnp.float32))
        if want_v:
            out_shapes.append(jax.ShapeDtypeStruct((1, n), jnp.float32))
    body = functools.partial(_projs_body, specs)
    return pl.pallas_call(body, out_shape=out_shapes)(*operands)


def _hbs_body(A_ref, u_ref, vt_ref, m_ref, o_ref):
    A = A_ref[...]
    e = _leaky(u_ref[...] + vt_ref[...], 0.2)
    e = jnp.where(A != 0, e, NEG)
    mx = jnp.max(e, axis=1, keepdims=True)
    g = jnp.max(mx)
    z = A * jnp.exp(e - g)
    w = z * jnp.exp(g - mx)
    s = jnp.maximum(jnp.sum(w, axis=1, keepdims=True), 1e-13)
    o = jnp.dot(w, m_ref[...], preferred_element_type=jnp.float32)
    o_ref[...] = jnp.maximum(o / s, 0.0)


def _hbs_attn(A, u, vt, m, tile):
    N, S = A.shape
    return pl.pallas_call(
        _hbs_body,
        grid=(N // tile,),
        in_specs=[
            pl.BlockSpec((tile, S), lambda i: (i, 0)),
            pl.BlockSpec((tile, 1), lambda i: (i, 0)),
            pl.BlockSpec((1, S), lambda i: (0, 0)),
            pl.BlockSpec((S, C), lambda i: (0, 0)),
        ],
        out_specs=pl.BlockSpec((tile, C), lambda i: (i, 0)),
        out_shape=jax.ShapeDtypeStruct((N, C), jnp.float32),
    )(A, u, vt, m)


def _hbns_body(A_ref, u_ref, vt_ref, sm_ref, tm_ref, ot_ref, os_ref,
               cmax_ref, csum_ref, acc_ref):
    i = pl.program_id(0)
    nt = pl.num_programs(0)
    A = A_ref[...]
    e = _leaky(u_ref[...] + vt_ref[...], 0.2)
    e = jnp.where(A != 0, e, NEG)
    mx = jnp.max(e, axis=1, keepdims=True)
    g = jnp.max(mx)
    z = A * jnp.exp(e - g)

    w1 = z * jnp.exp(g - mx)
    s = jnp.maximum(jnp.sum(w1, axis=1, keepdims=True), 1e-13)
    o = jnp.dot(w1, sm_ref[...], preferred_element_type=jnp.float32)
    ot_ref[...] = jnp.maximum(o / s, 0.0)

    @pl.when(i == 0)
    def _():
        cmax_ref[...] = jnp.full(cmax_ref.shape, NEG, jnp.float32)
        csum_ref[...] = jnp.zeros(csum_ref.shape, jnp.float32)
        acc_ref[...] = jnp.zeros(acc_ref.shape, jnp.float32)

    tmx = jnp.max(e, axis=0, keepdims=True)
    new = jnp.maximum(cmax_ref[...], tmx)
    scale = jnp.exp(cmax_ref[...] - new)
    w2 = z * jnp.exp(g - new)
    csum_ref[...] = csum_ref[...] * scale + jnp.sum(w2, axis=0, keepdims=True)
    pacc = jax.lax.dot_general(
        w2, tm_ref[...], (((0,), (0,)), ((), ())),
        preferred_element_type=jnp.float32)
    acc_ref[...] = acc_ref[...] * scale.T + pacc
    cmax_ref[...] = new

    @pl.when(i == nt - 1)
    def _():
        den = jnp.maximum(csum_ref[...], 1e-13)
        os_ref[...] = jnp.maximum(acc_ref[...] / den.T, 0.0)


def _hbns_attn(A, u, vt, sm, tm, tile):
    T, S = A.shape
    return pl.pallas_call(
        _hbns_body,
        grid=(T // tile,),
        in_specs=[
            pl.BlockSpec((tile, S), lambda i: (i, 0)),
            pl.BlockSpec((tile, 1), lambda i: (i, 0)),
            pl.BlockSpec((1, S), lambda i: (0, 0)),
            pl.BlockSpec((S, C), lambda i: (0, 0)),
            pl.BlockSpec((tile, C), lambda i: (i, 0)),
        ],
        out_specs=[
            pl.BlockSpec((tile, C), lambda i: (i, 0)),
            pl.BlockSpec((S, C), lambda i: (0, 0)),
        ],
        out_shape=[
            jax.ShapeDtypeStruct((T, C), jnp.float32),
            jax.ShapeDtypeStruct((S, C), jnp.float32),
        ],
        scratch_shapes=[
            pltpu.VMEM((1, S), jnp.float32),
            pltpu.VMEM((1, S), jnp.float32),
            pltpu.VMEM((S, C), jnp.float32),
        ],
    )(A, u, vt, sm, tm)


def _head_body(a0_ref, b0_ref, a1_ref, b1_ref, c1_ref, a2_ref, b2_ref,
               w1_ref, bb1_ref, w2_ref, bb2_ref, w3_ref, bb3_ref,
               w4_ref, bb4_ref, o_ref):
    x0 = jnp.maximum(a0_ref[...] + b0_ref[...], 0.0)
    p0 = jnp.max(x0, axis=0, keepdims=True)
    x1 = jnp.maximum(a1_ref[...] + b1_ref[...] + c1_ref[...], 0.0)
    p1 = jnp.max(x1, axis=0, keepdims=True)
    x2 = jnp.maximum(a2_ref[...] + b2_ref[...], 0.0)
    p2 = jnp.max(x2, axis=0, keepdims=True)
    w1 = w1_ref[...]
    h = (jnp.dot(p0, w1[0:C], preferred_element_type=jnp.float32)
         + jnp.dot(p1, w1[C:2 * C], preferred_element_type=jnp.float32)
         + jnp.dot(p2, w1[2 * C:3 * C], preferred_element_type=jnp.float32)
         + bb1_ref[...])
    h = _leaky(h, 0.01)
    h = _leaky(jnp.dot(h, w2_ref[...], preferred_element_type=jnp.float32)
               + bb2_ref[...], 0.01)
    h = _leaky(jnp.dot(h, w3_ref[...], preferred_element_type=jnp.float32)
               + bb3_ref[...], 0.01)
    o_ref[...] = (jnp.dot(h, w4_ref[...], preferred_element_type=jnp.float32)
                  + bb4_ref[...])


def _head(msgs, p):
    out_dim = p["fc4_b"].shape[0]
    return pl.pallas_call(
        _head_body,
        out_shape=jax.ShapeDtypeStruct((1, out_dim), jnp.float32),
    )(*msgs,
      p["fc1_w"], p["fc1_b"].reshape(1, -1),
      p["fc2_w"], p["fc2_b"].reshape(1, -1),
      p["fc3_w"], p["fc3_b"].reshape(1, -1),
      p["fc4_w"], p["fc4_b"].reshape(1, -1))


def _halves(a):
    return a[:C].reshape(C, 1), a[C:].reshape(C, 1)


def kernel(x_0, x_1, x_2, neighborhood_0_to_0, neighborhood_1_to_1,
           neighborhood_2_to_2, neighborhood_0_to_1, neighborhood_1_to_2,
           params):
    p = params
    n00, n11, n22 = neighborhood_0_to_0, neighborhood_1_to_1, neighborhood_2_to_2
    n01, n12 = neighborhood_0_to_1, neighborhood_1_to_2
    TT = 200

    au0, av0 = _halves(p["hbs0_l1_a"])
    asrc01, atgt01 = _halves(p["hbns01_l1_a"])
    asrc12, atgt12 = _halves(p["hbns12_l1_a"])
    (m0, u0, v0t,
     tm01, ut01,
     sm01, vst01,
     tm12, ut12,
     sm12, vst12) = _projs([
        ((x_0,), p["hbs0_l1_w"], au0, av0, False, True, True),
        ((x_0,), p["hbns01_l1_wt"], atgt01, atgt01, False, True, False),
        ((x_1,), p["hbns01_l1_ws"], asrc01, asrc01, False, False, True),
        ((x_1,), p["hbns12_l1_wt"], atgt12, atgt12, False, True, False),
        ((x_2,), p["hbns12_l1_ws"], asrc12, asrc12, False, False, True),
    ])
    x00 = _hbs_attn(n00, u0, v0t, m0, TT)
    x1to0, x0to1 = _hbns_attn(n01, ut01, vst01, sm01, tm01, TT)
    x2to1, x1to2 = _hbns_attn(n12, ut12, vst12, sm12, tm12, TT)

    x0l1 = (x00, x1to0)
    x1l1 = (x0to1, x2to1)
    x2l1 = (x1to2,)

    au0b, av0b = _halves(p["hbs0_l2_a"])
    asrc01b, atgt01b = _halves(p["hbns01_l2_a"])
    au1b, av1b = _halves(p["hbs1_l2_a"])
    asrc12b, atgt12b = _halves(p["hbns12_l2_a"])
    au2b, av2b = _halves(p["hbs2_l2_a"])
    (m0b, u0b, v0bt,
     tm01b, ut01b,
     sm01b, vst01b,
     m1b, u1b, v1bt,
     tm12b, ut12b,
     sm12b, vst12b,
     m2b, u2b, v2bt) = _projs([
        (x0l1, p["hbs0_l2_w"], au0b, av0b, True, True, True),
        (x0l1, p["hbns01_l2_wt"], atgt01b, atgt01b, True, True, False),
        (x1l1, p["hbns01_l2_ws"], asrc01b, asrc01b, True, False, True),
        (x1l1, p["hbs1_l2_w"], au1b, av1b, True, True, True),
        (x1l1, p["hbns12_l2_wt"], atgt12b, atgt12b, True, True, False),
        (x2l1, p["hbns12_l2_ws"], asrc12b, asrc12b, True, False, True),
        (x2l1, p["hbs2_l2_w"], au2b, av2b, True, True, True),
    ])
    x00_2 = _hbs_attn(n00, u0b, v0bt, m0b, TT)
    x1to0_2, x0to1_2 = _hbns_attn(n01, ut01b, vst01b, sm01b, tm01b, TT)
    x11 = _hbs_attn(n11, u1b, v1bt, m1b, TT)
    x2to1_2, x1to2_2 = _hbns_attn(n12, ut12b, vst12b, sm12b, tm12b, TT)
    x22 = _hbs_attn(n22, u2b, v2bt, m2b, TT)

    return _head((x00_2, x1to0_2, x0to1_2, x11, x2to1_2, x1to2_2, x22), p)


# speedup vs baseline: 1.8963x; 1.2681x over previous
import functools

import jax
import jax.numpy as jnp
from jax.experimental import pallas as pl
from jax.experimental.pallas import tpu as pltpu

C = 128
CLAMP = 60.0


def _leaky(x, slope):
    return jnp.where(x > 0, x, slope * x)


def _projs_body(specs, *refs):
    pos = 0
    ins = []
    for n_add, relu_in, want_u, want_v in specs:
        ins.append(refs[pos:pos + n_add + 3])
        pos += n_add + 3
    outs = refs[pos:]
    opos = 0
    for (n_add, relu_in, want_u, want_v), group in zip(specs, ins):
        xs = group[:n_add]
        w_ref, au_ref, av_ref = group[n_add:]
        x = xs[0][...]
        for r in xs[1:]:
            x = x + r[...]
        if relu_in:
            x = jnp.maximum(x, 0.0)
        m = jnp.dot(x, w_ref[...], preferred_element_type=jnp.float32)
        outs[opos][...] = m.astype(jnp.bfloat16)
        opos += 1
        if want_u:
            u = jnp.dot(m, au_ref[...], preferred_element_type=jnp.float32)
            outs[opos][...] = u
            outs[opos + 1][...] = jnp.max(u).reshape(1, 1)
            opos += 2
        if want_v:
            vt = jax.lax.dot_general(
                av_ref[...], m, (((0,), (1,)), ((), ())),
                preferred_element_type=jnp.float32)
            outs[opos][...] = vt
            outs[opos + 1][...] = jnp.max(vt).reshape(1, 1)
            opos += 2


def _projs(groups):
    specs = tuple((len(xs), relu_in, want_u, want_v)
                  for xs, _, _, _, relu_in, want_u, want_v in groups)
    operands = []
    out_shapes = []
    for xs, w, au, av, relu_in, want_u, want_v in groups:
        n = xs[0].shape[0]
        operands.extend(xs)
        operands.extend((w, au, av))
        out_shapes.append(jax.ShapeDtypeStruct((n, C), jnp.bfloat16))
        if want_u:
            out_shapes.append(jax.ShapeDtypeStruct((n, 1), jnp.float32))
            out_shapes.append(jax.ShapeDtypeStruct((1, 1), jnp.float32))
        if want_v:
            out_shapes.append(jax.ShapeDtypeStruct((1, n), jnp.float32))
            out_shapes.append(jax.ShapeDtypeStruct((1, 1), jnp.float32))
    body = functools.partial(_projs_body, specs)
    return pl.pallas_call(body, out_shape=out_shapes)(*operands)


def _hbs_body(A_ref, u_ref, vt_ref, umax_ref, vmax_ref, m_ref, o_ref):
    um = umax_ref[0, 0]
    vm = vmax_ref[0, 0]
    G = jnp.maximum(um + vm, 0.0)
    u = u_ref[...]
    rowfac = jnp.exp(jnp.minimum(G - jnp.maximum(u + vm, 0.0), CLAMP))
    e = _leaky(u + vt_ref[...], 0.2)
    w = (A_ref[...] * jnp.exp(e - G)) * rowfac
    s = jnp.maximum(jnp.sum(w, axis=1, keepdims=True), 1e-13)
    o = jnp.dot(w.astype(jnp.bfloat16), m_ref[...],
                preferred_element_type=jnp.float32)
    o_ref[...] = jnp.maximum(o / s, 0.0)


def _hbs_attn(A, u, vt, umax, vmax, m, tile):
    N, S = A.shape
    return pl.pallas_call(
        _hbs_body,
        grid=(N // tile,),
        in_specs=[
            pl.BlockSpec((tile, S), lambda i: (i, 0)),
            pl.BlockSpec((tile, 1), lambda i: (i, 0)),
            pl.BlockSpec((1, S), lambda i: (0, 0)),
            pl.BlockSpec((1, 1), lambda i: (0, 0)),
            pl.BlockSpec((1, 1), lambda i: (0, 0)),
            pl.BlockSpec((S, C), lambda i: (0, 0)),
        ],
        out_specs=pl.BlockSpec((tile, C), lambda i: (i, 0)),
        out_shape=jax.ShapeDtypeStruct((N, C), jnp.float32),
    )(A, u, vt, umax, vmax, m)


def _hbns_body(A_ref, u_ref, vt_ref, umax_ref, vmax_ref, sm_ref, tm_ref,
               ot_ref, os_ref, csum_ref, acc_ref):
    i = pl.program_id(0)
    nt = pl.num_programs(0)
    um = umax_ref[0, 0]
    vm = vmax_ref[0, 0]
    G = jnp.maximum(um + vm, 0.0)
    u = u_ref[...]
    vt = vt_ref[...]
    rowfac = jnp.exp(jnp.minimum(G - jnp.maximum(u + vm, 0.0), CLAMP))
    colfac = jnp.exp(jnp.minimum(G - jnp.maximum(vt + um, 0.0), CLAMP))
    e = _leaky(u + vt, 0.2)
    z = A_ref[...] * jnp.exp(e - G)

    w1 = z * rowfac
    s = jnp.maximum(jnp.sum(w1, axis=1, keepdims=True), 1e-13)
    o = jnp.dot(w1.astype(jnp.bfloat16), sm_ref[...],
                preferred_element_type=jnp.float32)
    ot_ref[...] = jnp.maximum(o / s, 0.0)

    @pl.when(i == 0)
    def _():
        csum_ref[...] = jnp.zeros(csum_ref.shape, jnp.float32)
        acc_ref[...] = jnp.zeros(acc_ref.shape, jnp.float32)

    w2 = z * colfac
    csum_ref[...] += jnp.sum(w2, axis=0, keepdims=True)
    acc_ref[...] += jax.lax.dot_general(
        w2.astype(jnp.bfloat16), tm_ref[...], (((0,), (0,)), ((), ())),
        preferred_element_type=jnp.float32)

    @pl.when(i == nt - 1)
    def _():
        den = jnp.maximum(csum_ref[...], 1e-13)
        os_ref[...] = jnp.maximum(acc_ref[...] / den.T, 0.0)


def _hbns_attn(A, u, vt, umax, vmax, sm, tm, tile):
    T, S = A.shape
    return pl.pallas_call(
        _hbns_body,
        grid=(T // tile,),
        in_specs=[
            pl.BlockSpec((tile, S), lambda i: (i, 0)),
            pl.BlockSpec((tile, 1), lambda i: (i, 0)),
            pl.BlockSpec((1, S), lambda i: (0, 0)),
            pl.BlockSpec((1, 1), lambda i: (0, 0)),
            pl.BlockSpec((1, 1), lambda i: (0, 0)),
            pl.BlockSpec((S, C), lambda i: (0, 0)),
            pl.BlockSpec((tile, C), lambda i: (i, 0)),
        ],
        out_specs=[
            pl.BlockSpec((tile, C), lambda i: (i, 0)),
            pl.BlockSpec((S, C), lambda i: (0, 0)),
        ],
        out_shape=[
            jax.ShapeDtypeStruct((T, C), jnp.float32),
            jax.ShapeDtypeStruct((S, C), jnp.float32),
        ],
        scratch_shapes=[
            pltpu.VMEM((1, S), jnp.float32),
            pltpu.VMEM((S, C), jnp.float32),
        ],
    )(A, u, vt, umax, vmax, sm, tm)


def _head_body(a0_ref, b0_ref, a1_ref, b1_ref, c1_ref, a2_ref, b2_ref,
               w1_ref, bb1_ref, w2_ref, bb2_ref, w3_ref, bb3_ref,
               w4_ref, bb4_ref, o_ref):
    x0 = jnp.maximum(a0_ref[...] + b0_ref[...], 0.0)
    p0 = jnp.max(x0, axis=0, keepdims=True)
    x1 = jnp.maximum(a1_ref[...] + b1_ref[...] + c1_ref[...], 0.0)
    p1 = jnp.max(x1, axis=0, keepdims=True)
    x2 = jnp.maximum(a2_ref[...] + b2_ref[...], 0.0)
    p2 = jnp.max(x2, axis=0, keepdims=True)
    w1 = w1_ref[...]
    h = (jnp.dot(p0, w1[0:C], preferred_element_type=jnp.float32)
         + jnp.dot(p1, w1[C:2 * C], preferred_element_type=jnp.float32)
         + jnp.dot(p2, w1[2 * C:3 * C], preferred_element_type=jnp.float32)
         + bb1_ref[...])
    h = _leaky(h, 0.01)
    h = _leaky(jnp.dot(h, w2_ref[...], preferred_element_type=jnp.float32)
               + bb2_ref[...], 0.01)
    h = _leaky(jnp.dot(h, w3_ref[...], preferred_element_type=jnp.float32)
               + bb3_ref[...], 0.01)
    o_ref[...] = (jnp.dot(h, w4_ref[...], preferred_element_type=jnp.float32)
                  + bb4_ref[...])


def _head(msgs, p):
    out_dim = p["fc4_b"].shape[0]
    return pl.pallas_call(
        _head_body,
        out_shape=jax.ShapeDtypeStruct((1, out_dim), jnp.float32),
    )(*msgs,
      p["fc1_w"], p["fc1_b"].reshape(1, -1),
      p["fc2_w"], p["fc2_b"].reshape(1, -1),
      p["fc3_w"], p["fc3_b"].reshape(1, -1),
      p["fc4_w"], p["fc4_b"].reshape(1, -1))


def _halves(a):
    return a[:C].reshape(C, 1), a[C:].reshape(C, 1)


def kernel(x_0, x_1, x_2, neighborhood_0_to_0, neighborhood_1_to_1,
           neighborhood_2_to_2, neighborhood_0_to_1, neighborhood_1_to_2,
           params):
    p = params
    n00, n11, n22 = neighborhood_0_to_0, neighborhood_1_to_1, neighborhood_2_to_2
    n01, n12 = neighborhood_0_to_1, neighborhood_1_to_2
    TT = 200

    au0, av0 = _halves(p["hbs0_l1_a"])
    asrc01, atgt01 = _halves(p["hbns01_l1_a"])
    asrc12, atgt12 = _halves(p["hbns12_l1_a"])
    (m0, u0, u0m, v0t, v0m,
     tm01, ut01, ut01m,
     sm01, vst01, vst01m,
     tm12, ut12, ut12m,
     sm12, vst12, vst12m) = _projs([
        ((x_0,), p["hbs0_l1_w"], au0, av0, False, True, True),
        ((x_0,), p["hbns01_l1_wt"], atgt01, atgt01, False, True, False),
        ((x_1,), p["hbns01_l1_ws"], asrc01, asrc01, False, False, True),
        ((x_1,), p["hbns12_l1_wt"], atgt12, atgt12, False, True, False),
        ((x_2,), p["hbns12_l1_ws"], asrc12, asrc12, False, False, True),
    ])
    x00 = _hbs_attn(n00, u0, v0t, u0m, v0m, m0, TT)
    x1to0, x0to1 = _hbns_attn(n01, ut01, vst01, ut01m, vst01m, sm01, tm01, TT)
    x2to1, x1to2 = _hbns_attn(n12, ut12, vst12, ut12m, vst12m, sm12, tm12, TT)

    x0l1 = (x00, x1to0)
    x1l1 = (x0to1, x2to1)
    x2l1 = (x1to2,)

    au0b, av0b = _halves(p["hbs0_l2_a"])
    asrc01b, atgt01b = _halves(p["hbns01_l2_a"])
    au1b, av1b = _halves(p["hbs1_l2_a"])
    asrc12b, atgt12b = _halves(p["hbns12_l2_a"])
    au2b, av2b = _halves(p["hbs2_l2_a"])
    (m0b, u0b, u0bm, v0bt, v0bm,
     tm01b, ut01b, ut01bm,
     sm01b, vst01b, vst01bm,
     m1b, u1b, u1bm, v1bt, v1bm,
     tm12b, ut12b, ut12bm,
     sm12b, vst12b, vst12bm,
     m2b, u2b, u2bm, v2bt, v2bm) = _projs([
        (x0l1, p["hbs0_l2_w"], au0b, av0b, True, True, True),
        (x0l1, p["hbns01_l2_wt"], atgt01b, atgt01b, True, True, False),
        (x1l1, p["hbns01_l2_ws"], asrc01b, asrc01b, True, False, True),
        (x1l1, p["hbs1_l2_w"], au1b, av1b, True, True, True),
        (x1l1, p["hbns12_l2_wt"], atgt12b, atgt12b, True, True, False),
        (x2l1, p["hbns12_l2_ws"], asrc12b, asrc12b, True, False, True),
        (x2l1, p["hbs2_l2_w"], au2b, av2b, True, True, True),
    ])
    x00_2 = _hbs_attn(n00, u0b, v0bt, u0bm, v0bm, m0b, TT)
    x1to0_2, x0to1_2 = _hbns_attn(n01, ut01b, vst01b, ut01bm, vst01bm,
                                  sm01b, tm01b, TT)
    x11 = _hbs_attn(n11, u1b, v1bt, u1bm, v1bm, m1b, TT)
    x2to1_2, x1to2_2 = _hbns_attn(n12, ut12b, vst12b, ut12bm, vst12bm,
                                  sm12b, tm12b, TT)
    x22 = _hbs_attn(n22, u2b, v2bt, u2bm, v2bm, m2b, TT)

    return _head((x00_2, x1to0_2, x0to1_2, x11, x2to1_2, x1to2_2, x22), p)


# speedup vs baseline: 2.1272x; 1.1218x over previous
import functools

import jax
import jax.numpy as jnp
from jax.experimental import pallas as pl
from jax.experimental.pallas import tpu as pltpu

C = 128
CLAMP = 60.0


def _leaky(x, slope):
    return jnp.where(x > 0, x, slope * x)


def _projs_body(specs, *refs):
    pos = 0
    ins = []
    for n_add, relu_in, want_u, want_v in specs:
        ins.append(refs[pos:pos + n_add + 3])
        pos += n_add + 3
    outs = refs[pos:]
    opos = 0
    for (n_add, relu_in, want_u, want_v), group in zip(specs, ins):
        xs = group[:n_add]
        w_ref, au_ref, av_ref = group[n_add:]
        x = xs[0][...]
        for r in xs[1:]:
            x = x + r[...]
        if relu_in:
            x = jnp.maximum(x, 0.0)
        m = jnp.dot(x, w_ref[...], preferred_element_type=jnp.float32)
        outs[opos][...] = m.astype(jnp.bfloat16)
        opos += 1
        if want_u:
            u = jnp.dot(m, au_ref[...], preferred_element_type=jnp.float32)
            outs[opos][...] = u
            outs[opos + 1][...] = jnp.max(u).reshape(1, 1)
            opos += 2
        if want_v:
            vt = jax.lax.dot_general(
                av_ref[...], m, (((0,), (1,)), ((), ())),
                preferred_element_type=jnp.float32)
            outs[opos][...] = vt
            outs[opos + 1][...] = jnp.max(vt).reshape(1, 1)
            opos += 2


def _projs(groups):
    specs = tuple((len(xs), relu_in, want_u, want_v)
                  for xs, _, _, _, relu_in, want_u, want_v in groups)
    operands = []
    out_shapes = []
    for xs, w, au, av, relu_in, want_u, want_v in groups:
        n = xs[0].shape[0]
        operands.extend(xs)
        operands.extend((w, au, av))
        out_shapes.append(jax.ShapeDtypeStruct((n, C), jnp.bfloat16))
        if want_u:
            out_shapes.append(jax.ShapeDtypeStruct((n, 1), jnp.float32))
            out_shapes.append(jax.ShapeDtypeStruct((1, 1), jnp.float32))
        if want_v:
            out_shapes.append(jax.ShapeDtypeStruct((1, n), jnp.float32))
            out_shapes.append(jax.ShapeDtypeStruct((1, 1), jnp.float32))
    body = functools.partial(_projs_body, specs)
    return pl.pallas_call(body, out_shape=out_shapes)(*operands)


def _hbs_body(A_ref, u_ref, vt_ref, umax_ref, vmax_ref, m_ref, o_ref):
    um = umax_ref[0, 0]
    vm = vmax_ref[0, 0]
    G = jnp.maximum(um + vm, 0.0)
    u = u_ref[...]
    rowfac = jnp.exp(jnp.minimum(G - jnp.maximum(u + vm, 0.0), CLAMP))
    e = _leaky(u + vt_ref[...], 0.2)
    S = A_ref.shape[1]
    wb = ((A_ref[...] * jnp.exp(e - G)) * rowfac).astype(jnp.bfloat16)
    s = jnp.dot(wb, jnp.ones((S, 1), jnp.bfloat16),
                preferred_element_type=jnp.float32)
    s = jnp.maximum(s, 1e-13)
    o = jnp.dot(wb, m_ref[...], preferred_element_type=jnp.float32)
    o_ref[...] = jnp.maximum(o / s, 0.0)


def _hbs_attn(A, u, vt, umax, vmax, m, tile):
    N, S = A.shape
    return pl.pallas_call(
        _hbs_body,
        grid=(N // tile,),
        in_specs=[
            pl.BlockSpec((tile, S), lambda i: (i, 0)),
            pl.BlockSpec((tile, 1), lambda i: (i, 0)),
            pl.BlockSpec((1, S), lambda i: (0, 0)),
            pl.BlockSpec((1, 1), lambda i: (0, 0)),
            pl.BlockSpec((1, 1), lambda i: (0, 0)),
            pl.BlockSpec((S, C), lambda i: (0, 0)),
        ],
        out_specs=pl.BlockSpec((tile, C), lambda i: (i, 0)),
        out_shape=jax.ShapeDtypeStruct((N, C), jnp.float32),
    )(A, u, vt, umax, vmax, m)


def _hbns_body(A_ref, u_ref, vt_ref, umax_ref, vmax_ref, sm_ref, tm_ref,
               ot_ref, os_ref, csum_ref, acc_ref):
    i = pl.program_id(0)
    nt = pl.num_programs(0)
    um = umax_ref[0, 0]
    vm = vmax_ref[0, 0]
    G = jnp.maximum(um + vm, 0.0)
    u = u_ref[...]
    vt = vt_ref[...]
    rowfac = jnp.exp(jnp.minimum(G - jnp.maximum(u + vm, 0.0), CLAMP))
    colfac = jnp.exp(jnp.minimum(G - jnp.maximum(vt + um, 0.0), CLAMP))
    e = _leaky(u + vt, 0.2)
    z = A_ref[...] * jnp.exp(e - G)

    S = A_ref.shape[1]
    w1b = (z * rowfac).astype(jnp.bfloat16)
    s = jnp.dot(w1b, jnp.ones((S, 1), jnp.bfloat16),
                preferred_element_type=jnp.float32)
    s = jnp.maximum(s, 1e-13)
    o = jnp.dot(w1b, sm_ref[...], preferred_element_type=jnp.float32)
    ot_ref[...] = jnp.maximum(o / s, 0.0)

    @pl.when(i == 0)
    def _():
        csum_ref[...] = jnp.zeros(csum_ref.shape, jnp.float32)
        acc_ref[...] = jnp.zeros(acc_ref.shape, jnp.float32)

    w2b = (z * colfac).astype(jnp.bfloat16)
    csum_ref[...] += jnp.sum(w2b.astype(jnp.float32), axis=0, keepdims=True)
    acc_ref[...] += jax.lax.dot_general(
        w2b, tm_ref[...], (((0,), (0,)), ((), ())),
        preferred_element_type=jnp.float32)

    @pl.when(i == nt - 1)
    def _():
        den = jnp.maximum(csum_ref[...], 1e-13)
        os_ref[...] = jnp.maximum(acc_ref[...] / den.T, 0.0)


def _hbns_attn(A, u, vt, umax, vmax, sm, tm, tile):
    T, S = A.shape
    return pl.pallas_call(
        _hbns_body,
        grid=(T // tile,),
        in_specs=[
            pl.BlockSpec((tile, S), lambda i: (i, 0)),
            pl.BlockSpec((tile, 1), lambda i: (i, 0)),
            pl.BlockSpec((1, S), lambda i: (0, 0)),
            pl.BlockSpec((1, 1), lambda i: (0, 0)),
            pl.BlockSpec((1, 1), lambda i: (0, 0)),
            pl.BlockSpec((S, C), lambda i: (0, 0)),
            pl.BlockSpec((tile, C), lambda i: (i, 0)),
        ],
        out_specs=[
            pl.BlockSpec((tile, C), lambda i: (i, 0)),
            pl.BlockSpec((S, C), lambda i: (0, 0)),
        ],
        out_shape=[
            jax.ShapeDtypeStruct((T, C), jnp.float32),
            jax.ShapeDtypeStruct((S, C), jnp.float32),
        ],
        scratch_shapes=[
            pltpu.VMEM((1, S), jnp.float32),
            pltpu.VMEM((S, C), jnp.float32),
        ],
    )(A, u, vt, umax, vmax, sm, tm)


def _head_body(a0_ref, b0_ref, a1_ref, b1_ref, c1_ref, a2_ref, b2_ref,
               w1_ref, bb1_ref, w2_ref, bb2_ref, w3_ref, bb3_ref,
               w4_ref, bb4_ref, o_ref):
    x0 = jnp.maximum(a0_ref[...] + b0_ref[...], 0.0)
    p0 = jnp.max(x0, axis=0, keepdims=True)
    x1 = jnp.maximum(a1_ref[...] + b1_ref[...] + c1_ref[...], 0.0)
    p1 = jnp.max(x1, axis=0, keepdims=True)
    x2 = jnp.maximum(a2_ref[...] + b2_ref[...], 0.0)
    p2 = jnp.max(x2, axis=0, keepdims=True)
    w1 = w1_ref[...]
    h = (jnp.dot(p0, w1[0:C], preferred_element_type=jnp.float32)
         + jnp.dot(p1, w1[C:2 * C], preferred_element_type=jnp.float32)
         + jnp.dot(p2, w1[2 * C:3 * C], preferred_element_type=jnp.float32)
         + bb1_ref[...])
    h = _leaky(h, 0.01)
    h = _leaky(jnp.dot(h, w2_ref[...], preferred_element_type=jnp.float32)
               + bb2_ref[...], 0.01)
    h = _leaky(jnp.dot(h, w3_ref[...], preferred_element_type=jnp.float32)
               + bb3_ref[...], 0.01)
    o_ref[...] = (jnp.dot(h, w4_ref[...], preferred_element_type=jnp.float32)
                  + bb4_ref[...])


def _head(msgs, p):
    out_dim = p["fc4_b"].shape[0]
    return pl.pallas_call(
        _head_body,
        out_shape=jax.ShapeDtypeStruct((1, out_dim), jnp.float32),
    )(*msgs,
      p["fc1_w"], p["fc1_b"].reshape(1, -1),
      p["fc2_w"], p["fc2_b"].reshape(1, -1),
      p["fc3_w"], p["fc3_b"].reshape(1, -1),
      p["fc4_w"], p["fc4_b"].reshape(1, -1))


def _halves(a):
    return a[:C].reshape(C, 1), a[C:].reshape(C, 1)


def kernel(x_0, x_1, x_2, neighborhood_0_to_0, neighborhood_1_to_1,
           neighborhood_2_to_2, neighborhood_0_to_1, neighborhood_1_to_2,
           params):
    p = params
    n00, n11, n22 = neighborhood_0_to_0, neighborhood_1_to_1, neighborhood_2_to_2
    n01, n12 = neighborhood_0_to_1, neighborhood_1_to_2
    _tile = {1000: 200, 2000: 400, 3000: 600}

    au0, av0 = _halves(p["hbs0_l1_a"])
    asrc01, atgt01 = _halves(p["hbns01_l1_a"])
    asrc12, atgt12 = _halves(p["hbns12_l1_a"])
    (m0, u0, u0m, v0t, v0m,
     tm01, ut01, ut01m,
     sm01, vst01, vst01m,
     tm12, ut12, ut12m,
     sm12, vst12, vst12m) = _projs([
        ((x_0,), p["hbs0_l1_w"], au0, av0, False, True, True),
        ((x_0,), p["hbns01_l1_wt"], atgt01, atgt01, False, True, False),
        ((x_1,), p["hbns01_l1_ws"], asrc01, asrc01, False, False, True),
        ((x_1,), p["hbns12_l1_wt"], atgt12, atgt12, False, True, False),
        ((x_2,), p["hbns12_l1_ws"], asrc12, asrc12, False, False, True),
    ])
    x00 = _hbs_attn(n00, u0, v0t, u0m, v0m, m0, _tile[1000])
    x1to0, x0to1 = _hbns_attn(n01, ut01, vst01, ut01m, vst01m, sm01, tm01, _tile[1000])
    x2to1, x1to2 = _hbns_attn(n12, ut12, vst12, ut12m, vst12m, sm12, tm12, _tile[3000])

    x0l1 = (x00, x1to0)
    x1l1 = (x0to1, x2to1)
    x2l1 = (x1to2,)

    au0b, av0b = _halves(p["hbs0_l2_a"])
    asrc01b, atgt01b = _halves(p["hbns01_l2_a"])
    au1b, av1b = _halves(p["hbs1_l2_a"])
    asrc12b, atgt12b = _halves(p["hbns12_l2_a"])
    au2b, av2b = _halves(p["hbs2_l2_a"])
    (m0b, u0b, u0bm, v0bt, v0bm,
     tm01b, ut01b, ut01bm,
     sm01b, vst01b, vst01bm,
     m1b, u1b, u1bm, v1bt, v1bm,
     tm12b, ut12b, ut12bm,
     sm12b, vst12b, vst12bm,
     m2b, u2b, u2bm, v2bt, v2bm) = _projs([
        (x0l1, p["hbs0_l2_w"], au0b, av0b, True, True, True),
        (x0l1, p["hbns01_l2_wt"], atgt01b, atgt01b, True, True, False),
        (x1l1, p["hbns01_l2_ws"], asrc01b, asrc01b, True, False, True),
        (x1l1, p["hbs1_l2_w"], au1b, av1b, True, True, True),
        (x1l1, p["hbns12_l2_wt"], atgt12b, atgt12b, True, True, False),
        (x2l1, p["hbns12_l2_ws"], asrc12b, asrc12b, True, False, True),
        (x2l1, p["hbs2_l2_w"], au2b, av2b, True, True, True),
    ])
    x00_2 = _hbs_attn(n00, u0b, v0bt, u0bm, v0bm, m0b, _tile[1000])
    x1to0_2, x0to1_2 = _hbns_attn(n01, ut01b, vst01b, ut01bm, vst01bm,
                                  sm01b, tm01b, _tile[1000])
    x11 = _hbs_attn(n11, u1b, v1bt, u1bm, v1bm, m1b, _tile[3000])
    x2to1_2, x1to2_2 = _hbns_attn(n12, ut12b, vst12b, ut12bm, vst12bm,
                                  sm12b, tm12b, _tile[3000])
    x22 = _hbs_attn(n22, u2b, v2bt, u2bm, v2bm, m2b, _tile[2000])

    return _head((x00_2, x1to0_2, x0to1_2, x11, x2to1_2, x1to2_2, x22), p)
